# baseline (speedup 1.0000x reference)
import functools

import jax
import jax.numpy as jnp
from jax import lax
from jax.experimental import pallas as pl
from jax.experimental.pallas import tpu as pltpu
from jax.experimental.pallas import tpu_sc as plsc

BATCH = 16384
DIM = 64
WIDE = 128
LANES = 16
NUM_WORKERS = 32
B_PER_W = BATCH // NUM_WORKERS
CHUNK = 128
N_CHUNKS = B_PER_W // CHUNK
GROUPS_PER_CHUNK = CHUNK // LANES


def _body(ent_hbm, rel_hbm, hidx_hbm, ridx_hbm, tidx_hbm, out_hbm,
          hidx_v, ridx_v, tidx_v, hhalf_v, rhalf_v, thalf_v,
          hrows_v, rrows_v, trows_v, out_v, *sems):
    wid = lax.axis_index("s") * 2 + lax.axis_index("c")
    row0 = wid * N_CHUNKS
    base = wid * B_PER_W

    pltpu.sync_copy(hidx_hbm.at[pl.ds(row0, N_CHUNKS)], hidx_v)
    pltpu.sync_copy(ridx_hbm.at[pl.ds(row0, N_CHUNKS)], ridx_v)
    pltpu.sync_copy(tidx_hbm.at[pl.ds(row0, N_CHUNKS)], tidx_v)

    for src, dst in ((hidx_v, hhalf_v), (ridx_v, rhalf_v), (tidx_v, thalf_v)):
        for k in range(N_CHUNKS):
            for v in range(CHUNK // LANES):
                sl = pl.ds(v * LANES, LANES)
                dst[k, sl] = lax.shift_right_logical(src[k, sl], 1)

    lanes = lax.iota(jnp.int32, LANES)
    zero = jnp.zeros((LANES,), jnp.float32)
    half = jnp.full((LANES,), 0.5, jnp.float32)
    three_half = jnp.full((LANES,), 1.5, jnp.float32)
    magic = jnp.full((LANES,), 0x5F3759DF, jnp.int32)
    six = jnp.int32(6)
    one = jnp.int32(1)

    def fire(k, slot):
        s = sems[slot]
        return (
            pltpu.async_copy(ent_hbm.at[hhalf_v.at[k]], hrows_v.at[slot], s),
            pltpu.async_copy(rel_hbm.at[rhalf_v.at[k]], rrows_v.at[slot], s),
            pltpu.async_copy(ent_hbm.at[thalf_v.at[k]], trows_v.at[slot], s),
        )

    def make_group(k, slot):
        def group(r, _):
            sl16 = pl.ds(r * LANES, LANES)
            hv = hidx_v[k, sl16]
            rv = ridx_v[k, sl16]
            tv = tidx_v[k, sl16]
            tot = zero
            for t in range(LANES):
                i = r * LANES + t
                ho = lax.shift_left(hv[t] & one, six)
                ro = lax.shift_left(rv[t] & one, six)
                to = lax.shift_left(tv[t] & one, six)
                acc = zero
                for j in range(DIM // LANES):
                    o = j * LANES
                    d = (hrows_v[slot, i, pl.ds(ho + o, LANES)]
                         + rrows_v[slot, i, pl.ds(ro + o, LANES)]
                         - trows_v[slot, i, pl.ds(to + o, LANES)])
                    acc = acc + d * d
                s = acc[0]
                for c in range(1, LANES):
                    s = s + acc[c]
                tot = jnp.where(lanes == t, s, tot)
            xi = lax.bitcast_convert_type(tot, jnp.int32)
            y = lax.bitcast_convert_type(
                magic - lax.shift_right_logical(xi, 1), jnp.float32)
            hx = half * tot
            for _ in range(3):
                y = y * (three_half - hx * y * y)
            out_v[pl.ds((k * GROUPS_PER_CHUNK + r) * LANES, LANES)] = tot * y
            return 0
        return group

    pending = fire(0, 0)
    for k in range(N_CHUNKS):
        nxt = fire(k + 1, (k + 1) % 2) if k + 1 < N_CHUNKS else None
        for c in pending:
            c.wait()
        lax.fori_loop(0, GROUPS_PER_CHUNK, make_group(k, k % 2), 0)
        pending = nxt

    pltpu.sync_copy(out_v, out_hbm.at[pl.ds(base, B_PER_W)])


@jax.jit
def kernel(triples, entity_table, relation_table):
    hidx = triples[:, 0].reshape(BATCH // CHUNK, CHUNK)
    ridx = triples[:, 1].reshape(BATCH // CHUNK, CHUNK)
    tidx = triples[:, 2].reshape(BATCH // CHUNK, CHUNK)
    ent2 = entity_table.reshape(-1, WIDE)
    rel2 = relation_table.reshape(-1, WIDE)

    run = functools.partial(
        pl.kernel,
        out_type=jax.ShapeDtypeStruct((BATCH,), jnp.float32),
        mesh=plsc.VectorSubcoreMesh(core_axis_name="c", subcore_axis_name="s"),
        scratch_types=[
            pltpu.VMEM((N_CHUNKS, CHUNK), jnp.int32),
            pltpu.VMEM((N_CHUNKS, CHUNK), jnp.int32),
            pltpu.VMEM((N_CHUNKS, CHUNK), jnp.int32),
            pltpu.VMEM((N_CHUNKS, CHUNK), jnp.int32),
            pltpu.VMEM((N_CHUNKS, CHUNK), jnp.int32),
            pltpu.VMEM((N_CHUNKS, CHUNK), jnp.int32),
            pltpu.VMEM((2, CHUNK, WIDE), jnp.float32),
            pltpu.VMEM((2, CHUNK, WIDE), jnp.float32),
            pltpu.VMEM((2, CHUNK, WIDE), jnp.float32),
            pltpu.VMEM((B_PER_W,), jnp.float32),
            pltpu.SemaphoreType.DMA,
            pltpu.SemaphoreType.DMA,
        ],
    )(_body)
    return run(ent2, rel2, hidx, ridx, tidx)
